# Initial kernel scaffold; baseline (speedup 1.0000x reference)
import jax
import jax.numpy as jnp
from jax.experimental import pallas as pl


def kernel(x, edge_index, W1a, b1a, g1, be1, W1b, b1b, W2a, b2a, g2, be2, W2b, b2b):
    raise NotImplementedError("write your pallas kernel here")


# speedup vs baseline: 4.1273x; 4.1273x over previous
import functools

import jax
import jax.numpy as jnp
from jax import lax
from jax.experimental import pallas as pl
from jax.experimental.pallas import tpu as pltpu
from jax.experimental.pallas import tpu_sc as plsc

_BN_EPS = 1e-5


def _pick_chunk(E, step):
    for ch in range(4096 - 4096 % step, 0, -step):
        if E % ch == 0:
            return ch
    return step


def _sc_aggregate(yT, src, dst):
    F, n = yT.shape
    E = src.shape[0]
    info = plsc.get_sparse_core_info()
    NC, NS, L = info.num_cores, info.num_subcores, info.num_lanes
    NW = NC * NS
    R = F // NW
    U = 8
    CH = _pick_chunk(E, L * U)
    mesh = plsc.VectorSubcoreMesh(core_axis_name="c", subcore_axis_name="s")

    @functools.partial(
        pl.kernel,
        mesh=mesh,
        compiler_params=pltpu.CompilerParams(needs_layout_passes=False),
        out_type=jax.ShapeDtypeStruct((F * n,), jnp.float32),
        scratch_types=[
            pltpu.VMEM((R * n,), jnp.float32),
            pltpu.VMEM((R * n,), jnp.float32),
            pltpu.VMEM((CH,), jnp.int32),
            pltpu.VMEM((CH,), jnp.int32),
            pltpu.SemaphoreType.DMA,
            pltpu.SemaphoreType.DMA,
        ],
    )
    def agg_kernel(yT_h, src_h, dst_h, out_h, ycol, acc, sbuf, dbuf, sem0, sem1):
        wid = lax.axis_index("s") * NC + lax.axis_index("c")
        e0 = wid * R * n
        cp_y = pltpu.async_copy(yT_h.at[pl.ds(e0, R * n)], ycol, sem0)
        cp_a = pltpu.async_copy(yT_h.at[pl.ds(e0, R * n)], acc, sem1)
        cp_y.wait()
        cp_a.wait()

        def chunk_body(kk, _):
            base = kk * CH
            c0 = pltpu.async_copy(src_h.at[pl.ds(base, CH)], sbuf, sem0)
            c1 = pltpu.async_copy(dst_h.at[pl.ds(base, CH)], dbuf, sem1)
            c0.wait()
            c1.wait()

            def vec_body(ii, _):
                b = ii * (L * U)
                for u in range(U):
                    off = b + u * L
                    si = sbuf[pl.ds(off, L)]
                    di = dbuf[pl.ds(off, L)]
                    for r in range(R):
                        sr = si if r == 0 else si + jnp.int32(r * n)
                        dr = di if r == 0 else di + jnp.int32(r * n)
                        v = plsc.load_gather(ycol, [sr])
                        plsc.addupdate_scatter(acc, [dr], v)
                return 0

            lax.fori_loop(0, CH // (L * U), vec_body, 0)
            return 0

        lax.fori_loop(0, E // CH, chunk_body, 0)
        pltpu.async_copy(acc, out_h.at[pl.ds(e0, R * n)], sem0).wait()

    out = agg_kernel(yT.reshape(F * n), src, dst)
    return out.reshape(F, n)


def _tc_in_proj(x, W1a):
    n, _ = x.shape
    f = W1a.shape[1]

    def body(x_ref, w_ref, o_ref):
        o_ref[...] = lax.dot_general(
            w_ref[...], x_ref[...], (((0,), (1,)), ((), ())),
            preferred_element_type=jnp.float32, precision=lax.Precision.HIGHEST)

    return pl.pallas_call(
        body, out_shape=jax.ShapeDtypeStruct((f, n), jnp.float32))(x, W1a)


def _tc_mid(agg1T, b1a, g1, be1, W1b, b1b, W2a):
    f1, n = agg1T.shape
    f2 = W2a.shape[1]

    def body(a_ref, ba_ref, g_ref, be_ref, wb_ref, bb_ref, wc_ref, o_ref):
        p = a_ref[...] + ba_ref[...]
        mu = jnp.mean(p, axis=1, keepdims=True)
        var = jnp.mean((p - mu) ** 2, axis=1, keepdims=True)
        bn = (p - mu) * lax.rsqrt(var + _BN_EPS) * g_ref[...] + be_ref[...]
        r = jnp.maximum(bn, 0.0)
        h = lax.dot_general(wb_ref[...], r, (((0,), (0,)), ((), ())),
                            preferred_element_type=jnp.float32, precision=lax.Precision.HIGHEST)
        z = jnp.maximum(h + bb_ref[...], 0.0)
        o_ref[...] = lax.dot_general(wc_ref[...], z, (((0,), (0,)), ((), ())),
                                     preferred_element_type=jnp.float32, precision=lax.Precision.HIGHEST)

    return pl.pallas_call(
        body, out_shape=jax.ShapeDtypeStruct((f2, n), jnp.float32))(
            agg1T, b1a, g1, be1, W1b, b1b, W2a)


def _tc_out(agg2T, b2a, g2, be2, W2b, b2b):
    _, n = agg2T.shape
    d = W2b.shape[1]

    def body(a_ref, ba_ref, g_ref, be_ref, wb_ref, bb_ref, h_ref, p_ref):
        p = a_ref[...] + ba_ref[...]
        mu = jnp.mean(p, axis=1, keepdims=True)
        var = jnp.mean((p - mu) ** 2, axis=1, keepdims=True)
        bn = (p - mu) * lax.rsqrt(var + _BN_EPS) * g_ref[...] + be_ref[...]
        r = jnp.maximum(bn, 0.0)
        h = lax.dot_general(r, wb_ref[...], (((0,), (0,)), ((), ())),
                            preferred_element_type=jnp.float32, precision=lax.Precision.HIGHEST)
        h = jnp.maximum(h + bb_ref[...], 0.0)
        h_ref[...] = h
        p_ref[...] = jnp.mean(h, axis=0, keepdims=True)

    return pl.pallas_call(
        body,
        out_shape=[jax.ShapeDtypeStruct((n, d), jnp.float32),
                   jax.ShapeDtypeStruct((1, d), jnp.float32)])(
            agg2T, b2a, g2, be2, W2b, b2b)


def kernel(x, edge_index, W1a, b1a, g1, be1, W1b, b1b,
           W2a, b2a, g2, be2, W2b, b2b):
    src = edge_index[0]
    dst = edge_index[1]
    y1T = _tc_in_proj(x, W1a)
    agg1T = _sc_aggregate(y1T, src, dst)
    y2T = _tc_mid(agg1T, b1a.reshape(-1, 1), g1.reshape(-1, 1),
                  be1.reshape(-1, 1), W1b, b1b.reshape(-1, 1), W2a)
    agg2T = _sc_aggregate(y2T, src, dst)
    h, pooled = _tc_out(agg2T, b2a.reshape(-1, 1), g2.reshape(-1, 1),
                        be2.reshape(-1, 1), W2b, b2b.reshape(1, -1))
    return (h, pooled)
